# Initial kernel scaffold; baseline (speedup 1.0000x reference)
import jax
import jax.numpy as jnp
from jax.experimental import pallas as pl


def kernel(node_features, adj_list, weight, bias):
    raise NotImplementedError("write your pallas kernel here")


# speedup vs baseline: 1.0469x; 1.0469x over previous
import jax
import jax.numpy as jnp
from jax.experimental import pallas as pl
from jax.experimental.pallas import tpu as pltpu

_BM = 200


def _rgcn_body(adj_ref, x_ref, w_ref, b_ref, o_ref):
    msg0 = jax.lax.dot(adj_ref[0], x_ref[...],
                       preferred_element_type=jnp.float32)
    msg1 = jax.lax.dot(adj_ref[1], x_ref[...],
                       preferred_element_type=jnp.float32)
    out = (jax.lax.dot(msg0, w_ref[0], preferred_element_type=jnp.float32)
           + jax.lax.dot(msg1, w_ref[1], preferred_element_type=jnp.float32)
           + b_ref[...])
    o_ref[...] = jnp.maximum(out, 0.0)


def kernel(node_features, adj_list, weight, bias):
    n, in_dim = node_features.shape
    r = adj_list.shape[0]
    out_dim = weight.shape[-1]
    num_m = n // _BM

    b2 = bias.reshape(1, out_dim)

    return pl.pallas_call(
        _rgcn_body,
        grid=(num_m,),
        in_specs=[
            pl.BlockSpec((r, _BM, n), lambda m: (0, m, 0)),
            pl.BlockSpec((n, in_dim), lambda m: (0, 0)),
            pl.BlockSpec((r, in_dim, out_dim), lambda m: (0, 0, 0)),
            pl.BlockSpec((1, out_dim), lambda m: (0, 0)),
        ],
        out_specs=pl.BlockSpec((_BM, out_dim), lambda m: (m, 0)),
        out_shape=jax.ShapeDtypeStruct((n, out_dim), jnp.float32),
        compiler_params=pltpu.CompilerParams(
            dimension_semantics=("arbitrary",),
        ),
    )(adj_list, node_features, weight, b2)
